# baseline (speedup 1.0000x reference)
import functools

import jax
import jax.numpy as jnp
from jax import lax
from jax.experimental import pallas as pl
from jax.experimental.pallas import tpu as pltpu
from jax.experimental.pallas import tpu_sc as plsc

_CONTACT_W = 2.5
_INSIDE_W = 1.0
_A1 = 0.005
_A2 = 0.005
_B1 = 1.0
_B2 = 0.04

_R = 512
_FAR = 1e7


def _tri_ids(t, nb):
    j = jnp.int32(0)
    for k in range(1, nb):
        tk = k * nb - k * (k - 1) // 2
        j = j + (t >= tk).astype(jnp.int32)
    tj = j * nb - j * (j - 1) // 2
    c = j + (t - tj)
    return j, c


def _contact_body(nb, nt, pts_ref, cols_ref, dbig_ref, w1_ref, w2_ref, out_ref,
                  cmin_ref, rmin_ref):
    t = pl.program_id(1)
    j, c = _tri_ids(t, nb)
    rows = pts_ref[0]
    ctile = cols_ref[0]

    @pl.when(t == 0)
    def _():
        cmin_ref[0:1, :] = jnp.full_like(cmin_ref[0:1, :], 3e38)
        rmin_ref[:, 0:1] = jnp.full_like(rmin_ref[:, 0:1], 3e38)

    sq_r = jnp.sum(rows * rows, axis=1, keepdims=True)
    sq_c = jnp.sum(ctile * ctile, axis=0, keepdims=True)
    crossm2 = jnp.dot(
        (rows * jnp.float32(-2.0)).astype(jnp.bfloat16),
        ctile.astype(jnp.bfloat16),
        preferred_element_type=jnp.float32,
    )
    acc = (sq_r + sq_c) + crossm2

    csl = pl.ds(c * _R, _R)

    @pl.when(j == c)
    def _():
        acc2 = acc + dbig_ref[...]
        pm = jnp.min(acc2, axis=0, keepdims=True)
        cmin_ref[0:1, csl] = jnp.minimum(cmin_ref[0:1, csl], pm)

    @pl.when(j != c)
    def _():
        pm0 = jnp.min(acc, axis=0, keepdims=True)
        cmin_ref[0:1, csl] = jnp.minimum(cmin_ref[0:1, csl], pm0)
        pm1 = jnp.min(acc, axis=1, keepdims=True)
        rsl = pl.ds(j * _R, _R)
        rmin_ref[rsl, 0:1] = jnp.minimum(rmin_ref[rsl, 0:1], pm1)

    @pl.when(t == nt - 1)
    def _():
        rm = jnp.transpose(rmin_ref[:, 0:1], (1, 0))
        m = jnp.minimum(cmin_ref[0:1, :], rm)
        v = jnp.sqrt(jnp.maximum(m, jnp.float32(1e-12)))
        t1 = jnp.tanh(v * jnp.float32(1.0 / _A2))
        t2 = jnp.tanh(v * jnp.float32(1.0 / _B2))
        lossv = w1_ref[0] * (t1 * t1) + w2_ref[0] * (t2 * t2)
        out_ref[0] = jnp.full((1, 128), jnp.sum(lossv), jnp.float32)


def _sc_sqrt(x):
    xi = lax.bitcast_convert_type(x, jnp.int32)
    y = lax.bitcast_convert_type(
        jnp.int32(0x5F3759DF) - lax.shift_right_logical(xi, 1), jnp.float32
    )
    hx = x * jnp.float32(0.5)
    for _ in range(3):
        t = (hx * y) * y
        y = y * (jnp.float32(1.5) - t)
    return x * y


def _fal_sc_body(vx, vy, vz, fa, fb, fc, fic_hbm, out_hbm,
                 fic_v, vid, crd, outv, sems):
    wid = lax.axis_index("s") * 2 + lax.axis_index("c")
    base = wid * 16
    half = fic_hbm.shape[0] // 2
    pltpu.sync_copy(fic_hbm.at[pl.ds(base, 16)], fic_v.at[pl.ds(0, 16)])
    pltpu.sync_copy(fic_hbm.at[pl.ds(half + base, 16)], fic_v.at[pl.ds(16, 16)])
    cps = [
        pltpu.async_copy(t.at[fic_v], vid.at[k], sems.at[k])
        for k, t in enumerate((fa, fb, fc))
    ]
    for cp in cps:
        cp.wait()
    cps = []
    for k in range(3):
        for d, t in enumerate((vx, vy, vz)):
            i = k * 3 + d
            cps.append(pltpu.async_copy(t.at[vid.at[k]], crd.at[i], sems.at[i]))
    for cp in cps:
        cp.wait()

    def coord(s, k, d):
        return crd[k * 3 + d, pl.ds(s * 16, 16)]

    comps = []
    norms = []
    for s in range(2):
        a = [coord(s, 0, d) for d in range(3)]
        e1 = [coord(s, 1, d) - a[d] for d in range(3)]
        e2 = [coord(s, 2, d) - a[d] for d in range(3)]
        nx = e1[1] * e2[2] - e1[2] * e2[1]
        ny = e1[2] * e2[0] - e1[0] * e2[2]
        nz = e1[0] * e2[1] - e1[1] * e2[0]
        comps.append((nx, ny, nz))
        norms.append(_sc_sqrt(nx * nx + ny * ny + nz * nz))
    d12 = (
        comps[0][0] * comps[1][0]
        + comps[0][1] * comps[1][1]
        + comps[0][2] * comps[1][2]
    )
    eps = jnp.float32(1e-12)
    dotn = d12 / ((norms[0] + eps) * (norms[1] + eps))
    total = jnp.float32(1.0) + dotn
    ids = lax.iota(jnp.int32, 16)
    for sh in (1, 2, 4, 8):
        rot = (ids + sh) & 15
        total = total + total.at[rot].get(mode="promise_in_bounds")
    outv[...] = total
    pltpu.sync_copy(outv, out_hbm.at[wid])


def kernel(vertices, faces, exterior, faces_in_contact):
    bs = vertices.shape[0]
    n_hd = exterior.shape[1]
    hd = vertices[:, ::3]
    np_ = ((n_hd + _R - 1) // _R) * _R
    nb = np_ // _R

    pts = jnp.pad(hd, ((0, 0), (0, np_ - n_hd), (0, 0)), constant_values=_FAR)
    cols = jnp.transpose(pts, (0, 2, 1))
    ext_f = exterior.astype(jnp.float32)
    w1 = jnp.float32(_CONTACT_W * _A1) * ext_f
    w2 = jnp.float32(_INSIDE_W * _B1) * (1.0 - ext_f)
    w1 = jnp.pad(w1, ((0, 0), (0, np_ - n_hd)))[:, None, :]
    w2 = jnp.pad(w2, ((0, 0), (0, np_ - n_hd)))[:, None, :]
    dbig = jnp.where(
        jnp.arange(_R, dtype=jnp.int32)[:, None]
        == jnp.arange(_R, dtype=jnp.int32)[None, :],
        jnp.float32(1e10),
        jnp.float32(0.0),
    )
    nt = nb * (nb + 1) // 2

    def _jmap(t):
        j = jnp.int32(0)
        for k in range(1, nb):
            j = j + (t >= k * nb - k * (k - 1) // 2).astype(jnp.int32)
        return j

    def _cmap(t):
        j = _jmap(t)
        return j + (t - (j * nb - j * (j - 1) // 2))

    contact_out = pl.pallas_call(
        functools.partial(_contact_body, nb, nt),
        grid=(bs, nt),
        in_specs=[
            pl.BlockSpec((1, _R, 3), lambda b, t: (b, _jmap(t), 0)),
            pl.BlockSpec((1, 3, _R), lambda b, t: (b, 0, _cmap(t))),
            pl.BlockSpec((_R, _R), lambda b, t: (0, 0)),
            pl.BlockSpec((1, 1, np_), lambda b, t: (b, 0, 0)),
            pl.BlockSpec((1, 1, np_), lambda b, t: (b, 0, 0)),
        ],
        out_specs=pl.BlockSpec((1, 1, 128), lambda b, t: (b, 0, 0)),
        out_shape=jax.ShapeDtypeStruct((bs, 1, 128), jnp.float32),
        scratch_shapes=[
            pltpu.VMEM((8, np_), jnp.float32),
            pltpu.VMEM((np_, 1), jnp.float32),
        ],
        compiler_params=pltpu.CompilerParams(
            dimension_semantics=("arbitrary", "arbitrary")
        ),
    )(pts, cols, dbig, w1, w2)
    contactloss = contact_out[:, 0, 0]

    vcols = [vertices[bs - 1, :, d] for d in range(3)]
    fcols = [faces[:, k] for k in range(3)]
    fic_flat = faces_in_contact[bs - 1].reshape(-1)

    mesh = plsc.VectorSubcoreMesh(
        core_axis_name="c", subcore_axis_name="s", num_cores=2, num_subcores=16
    )
    fal_parts = pl.kernel(
        _fal_sc_body,
        out_type=jax.ShapeDtypeStruct((32, 16), jnp.float32),
        mesh=mesh,
        scratch_types=[
            pltpu.VMEM((32,), jnp.int32),
            pltpu.VMEM((3, 32), jnp.int32),
            pltpu.VMEM((9, 32), jnp.float32),
            pltpu.VMEM((16,), jnp.float32),
            pltpu.SemaphoreType.DMA((9,)),
        ],
    )(*vcols, *fcols, fic_flat)
    fal = jnp.sum(fal_parts[:, 0])
    face_angle_loss = jnp.zeros((bs,), dtype=vertices.dtype).at[bs - 1].set(fal)
    return (contactloss, face_angle_loss)


# speedup vs baseline: 1.0935x; 1.0935x over previous
import functools

import jax
import jax.numpy as jnp
from jax import lax
from jax.experimental import pallas as pl
from jax.experimental.pallas import tpu as pltpu
from jax.experimental.pallas import tpu_sc as plsc

_CONTACT_W = 2.5
_INSIDE_W = 1.0
_A1 = 0.005
_A2 = 0.005
_B1 = 1.0
_B2 = 0.04

_R = 512
_FAR = 1e7


def _contact_body(nb, pts_ref, cols_ref, j_ref, enc_ref, out_ref, min_ref):
    j = pl.program_id(1)
    rows = pts_ref[0]
    cols = cols_ref[0]

    sq_r = jnp.sum(rows * rows, axis=1, keepdims=True)
    sq_c = jnp.sum(cols * cols, axis=0, keepdims=True)
    crossm2 = jnp.dot(
        (rows * jnp.float32(-2.0)).astype(jnp.bfloat16),
        cols.astype(jnp.bfloat16),
        preferred_element_type=jnp.float32,
    )
    acc = (sq_r + sq_c) + crossm2

    acc = jnp.where(j_ref[...] == j * _R, jnp.float32(1e10), acc)
    pm = jnp.min(acc, axis=0, keepdims=True)

    @pl.when(j == 0)
    def _():
        min_ref[0:1, :] = pm

    @pl.when(j > 0)
    def _():
        min_ref[0:1, :] = jnp.minimum(min_ref[0:1, :], pm)

    @pl.when(j == nb - 1)
    def _():
        m = min_ref[0:1, :]
        v = jnp.sqrt(jnp.maximum(m, jnp.float32(1e-12)))
        t1 = jnp.tanh(v * jnp.float32(1.0 / _A2))
        t2 = jnp.tanh(v * jnp.float32(1.0 / _B2))
        enc = enc_ref[0]
        lossv = jnp.where(
            enc == jnp.float32(1.0),
            jnp.float32(_CONTACT_W * _A1) * (t1 * t1),
            jnp.where(enc == jnp.float32(0.0),
                      jnp.float32(_INSIDE_W * _B1) * (t2 * t2),
                      jnp.float32(0.0)),
        )
        out_ref[0] = jnp.full((1, 128), jnp.sum(lossv), jnp.float32)


def _sc_sqrt(x):
    xi = lax.bitcast_convert_type(x, jnp.int32)
    y = lax.bitcast_convert_type(
        jnp.int32(0x5F3759DF) - lax.shift_right_logical(xi, 1), jnp.float32
    )
    hx = x * jnp.float32(0.5)
    for _ in range(3):
        t = (hx * y) * y
        y = y * (jnp.float32(1.5) - t)
    return x * y


def _fal_sc_body(vflat, fflat, fic_hbm, out_hbm,
                 fic_v, find, vtx, cind, crd, outv, sems):
    wid = lax.axis_index("s") * 2 + lax.axis_index("c")
    base = wid * 16
    half = fic_hbm.shape[0] // 2
    pltpu.sync_copy(fic_hbm.at[pl.ds(base, 16)], fic_v.at[pl.ds(0, 16)])
    pltpu.sync_copy(fic_hbm.at[pl.ds(half + base, 16)], fic_v.at[pl.ds(16, 16)])

    for h in range(2):
        f = fic_v[pl.ds(h * 16, 16)] * 3
        for k in range(3):
            find[k, pl.ds(h * 16, 16)] = f + k
    cps = [
        pltpu.async_copy(fflat.at[find.at[k]], vtx.at[k], sems.at[k])
        for k in range(3)
    ]
    for cp in cps:
        cp.wait()
    for k in range(3):
        for h in range(2):
            v3 = vtx[k, pl.ds(h * 16, 16)] * 3
            for d in range(3):
                cind[k * 3 + d, pl.ds(h * 16, 16)] = v3 + d
    cps = []
    for i in range(9):
        cps.append(pltpu.async_copy(vflat.at[cind.at[i]], crd.at[i], sems.at[i]))
    for cp in cps:
        cp.wait()

    def coord(s, k, d):
        return crd[k * 3 + d, pl.ds(s * 16, 16)]

    comps = []
    norms = []
    for s in range(2):
        a = [coord(s, 0, d) for d in range(3)]
        e1 = [coord(s, 1, d) - a[d] for d in range(3)]
        e2 = [coord(s, 2, d) - a[d] for d in range(3)]
        nx = e1[1] * e2[2] - e1[2] * e2[1]
        ny = e1[2] * e2[0] - e1[0] * e2[2]
        nz = e1[0] * e2[1] - e1[1] * e2[0]
        comps.append((nx, ny, nz))
        norms.append(_sc_sqrt(nx * nx + ny * ny + nz * nz))
    d12 = (
        comps[0][0] * comps[1][0]
        + comps[0][1] * comps[1][1]
        + comps[0][2] * comps[1][2]
    )
    eps = jnp.float32(1e-12)
    dotn = d12 / ((norms[0] + eps) * (norms[1] + eps))
    total = jnp.float32(1.0) + dotn
    ids = lax.iota(jnp.int32, 16)
    for sh in (1, 2, 4, 8):
        rot = (ids + sh) & 15
        total = total + total.at[rot].get(mode="promise_in_bounds")
    outv[...] = total
    pltpu.sync_copy(outv, out_hbm.at[wid])


def kernel(vertices, faces, exterior, faces_in_contact):
    bs = vertices.shape[0]
    n_hd = exterior.shape[1]
    hd = vertices[:, ::3]
    np_ = ((n_hd + _R - 1) // _R) * _R
    nb = np_ // _R

    pts = jnp.pad(hd, ((0, 0), (0, np_ - n_hd), (0, 0)), constant_values=_FAR)
    cols = jnp.transpose(pts, (0, 2, 1))
    enc = jnp.pad(
        exterior.astype(jnp.float32), ((0, 0), (0, np_ - n_hd)),
        constant_values=-1.0,
    )[:, None, :]
    jmat = (
        jnp.arange(np_, dtype=jnp.int32)[None, :]
        - jnp.arange(_R, dtype=jnp.int32)[:, None]
    )

    contact_out = pl.pallas_call(
        functools.partial(_contact_body, nb),
        grid=(bs, nb),
        in_specs=[
            pl.BlockSpec((1, _R, 3), lambda b, j: (b, j, 0)),
            pl.BlockSpec((1, 3, np_), lambda b, j: (b, 0, 0)),
            pl.BlockSpec((_R, np_), lambda b, j: (0, 0)),
            pl.BlockSpec((1, 1, np_), lambda b, j: (b, 0, 0)),
        ],
        out_specs=pl.BlockSpec((1, 1, 128), lambda b, j: (b, 0, 0)),
        out_shape=jax.ShapeDtypeStruct((bs, 1, 128), jnp.float32),
        scratch_shapes=[pltpu.VMEM((8, np_), jnp.float32)],
        compiler_params=pltpu.CompilerParams(
            dimension_semantics=("arbitrary", "arbitrary")
        ),
    )(pts, cols, jmat, enc)
    contactloss = contact_out[:, 0, 0]

    vflat = vertices[bs - 1].reshape(-1)
    fflat = faces.reshape(-1)
    fic_flat = faces_in_contact[bs - 1].reshape(-1)

    mesh = plsc.VectorSubcoreMesh(
        core_axis_name="c", subcore_axis_name="s", num_cores=2, num_subcores=16
    )
    fal_parts = pl.kernel(
        _fal_sc_body,
        out_type=jax.ShapeDtypeStruct((32, 16), jnp.float32),
        mesh=mesh,
        scratch_types=[
            pltpu.VMEM((32,), jnp.int32),
            pltpu.VMEM((3, 32), jnp.int32),
            pltpu.VMEM((3, 32), jnp.int32),
            pltpu.VMEM((9, 32), jnp.int32),
            pltpu.VMEM((9, 32), jnp.float32),
            pltpu.VMEM((16,), jnp.float32),
            pltpu.SemaphoreType.DMA((9,)),
        ],
    )(vflat, fflat, fic_flat)
    fal = jnp.sum(fal_parts[:, 0])
    face_angle_loss = jnp.zeros((bs,), dtype=vertices.dtype).at[bs - 1].set(fal)
    return (contactloss, face_angle_loss)


# speedup vs baseline: 1.3208x; 1.2078x over previous
import functools

import jax
import jax.numpy as jnp
from jax import lax
from jax.experimental import pallas as pl
from jax.experimental.pallas import tpu as pltpu
from jax.experimental.pallas import tpu_sc as plsc

_CONTACT_W = 2.5
_INSIDE_W = 1.0
_A1 = 0.005
_A2 = 0.005
_B1 = 1.0
_B2 = 0.04

_R = 512
_FAR = 1e7


def _contact_body(nb, pts_ref, cols_ref, j_ref, enc_ref, out_ref, min_ref):
    j = pl.program_id(1)
    rows = pts_ref[0]
    cols = cols_ref[0]

    sq_r = jnp.sum(rows * rows, axis=1, keepdims=True)
    sq_c = jnp.sum(cols * cols, axis=0, keepdims=True)
    crossm2 = jnp.dot(
        (rows * jnp.float32(-2.0)).astype(jnp.bfloat16),
        cols.astype(jnp.bfloat16),
        preferred_element_type=jnp.float32,
    )
    acc = (sq_r + sq_c) + crossm2

    acc = jnp.where(j_ref[...] == j * _R, jnp.float32(1e10), acc)
    pm = jnp.min(acc, axis=0, keepdims=True)

    @pl.when(j == 0)
    def _():
        min_ref[0:1, :] = pm

    @pl.when(j > 0)
    def _():
        min_ref[0:1, :] = jnp.minimum(min_ref[0:1, :], pm)

    @pl.when(j == nb - 1)
    def _():
        m = min_ref[0:1, :]
        v = jnp.sqrt(jnp.maximum(m, jnp.float32(1e-12)))
        t1 = jnp.tanh(v * jnp.float32(1.0 / _A2))
        t2 = jnp.tanh(v * jnp.float32(1.0 / _B2))
        enc = enc_ref[0]
        lossv = jnp.where(
            enc == jnp.float32(1.0),
            jnp.float32(_CONTACT_W * _A1) * (t1 * t1),
            jnp.where(enc == jnp.float32(0.0),
                      jnp.float32(_INSIDE_W * _B1) * (t2 * t2),
                      jnp.float32(0.0)),
        )
        out_ref[0] = jnp.full((1, 128), jnp.sum(lossv), jnp.float32)


def _sc_sqrt(x):
    xi = lax.bitcast_convert_type(x, jnp.int32)
    y = lax.bitcast_convert_type(
        jnp.int32(0x5F3759DF) - lax.shift_right_logical(xi, 1), jnp.float32
    )
    hx = x * jnp.float32(0.5)
    for _ in range(3):
        t = (hx * y) * y
        y = y * (jnp.float32(1.5) - t)
    return x * y


def _fal_sc_body(vx, vy, vz, fa, fb, fc, fic_hbm, out_hbm,
                 fic_v, vid, crd, outv, sems):
    wid = lax.axis_index("s") * 2 + lax.axis_index("c")
    base = wid * 16
    half = fic_hbm.shape[0] // 2
    pltpu.sync_copy(fic_hbm.at[pl.ds(base, 16)], fic_v.at[pl.ds(0, 16)])
    pltpu.sync_copy(fic_hbm.at[pl.ds(half + base, 16)], fic_v.at[pl.ds(16, 16)])
    cps = [
        pltpu.async_copy(t.at[fic_v], vid.at[k], sems.at[k])
        for k, t in enumerate((fa, fb, fc))
    ]
    for cp in cps:
        cp.wait()
    cps = []
    for k in range(3):
        for d, t in enumerate((vx, vy, vz)):
            i = k * 3 + d
            cps.append(pltpu.async_copy(t.at[vid.at[k]], crd.at[i], sems.at[i]))
    for cp in cps:
        cp.wait()

    def coord(s, k, d):
        return crd[k * 3 + d, pl.ds(s * 16, 16)]

    comps = []
    norms = []
    for s in range(2):
        a = [coord(s, 0, d) for d in range(3)]
        e1 = [coord(s, 1, d) - a[d] for d in range(3)]
        e2 = [coord(s, 2, d) - a[d] for d in range(3)]
        nx = e1[1] * e2[2] - e1[2] * e2[1]
        ny = e1[2] * e2[0] - e1[0] * e2[2]
        nz = e1[0] * e2[1] - e1[1] * e2[0]
        comps.append((nx, ny, nz))
        norms.append(_sc_sqrt(nx * nx + ny * ny + nz * nz))
    d12 = (
        comps[0][0] * comps[1][0]
        + comps[0][1] * comps[1][1]
        + comps[0][2] * comps[1][2]
    )
    eps = jnp.float32(1e-12)
    dotn = d12 / ((norms[0] + eps) * (norms[1] + eps))
    total = jnp.float32(1.0) + dotn
    ids = lax.iota(jnp.int32, 16)
    for sh in (1, 2, 4, 8):
        rot = (ids + sh) & 15
        total = total + total.at[rot].get(mode="promise_in_bounds")
    outv[...] = total
    pltpu.sync_copy(outv, out_hbm.at[wid])


def kernel(vertices, faces, exterior, faces_in_contact):
    bs = vertices.shape[0]
    n_hd = exterior.shape[1]
    hd = vertices[:, ::3]
    np_ = ((n_hd + _R - 1) // _R) * _R
    nb = np_ // _R

    pts = jnp.pad(hd, ((0, 0), (0, np_ - n_hd), (0, 0)), constant_values=_FAR)
    cols = jnp.transpose(pts, (0, 2, 1))
    enc = jnp.pad(
        exterior.astype(jnp.float32), ((0, 0), (0, np_ - n_hd)),
        constant_values=-1.0,
    )[:, None, :]
    jmat = (
        jnp.arange(np_, dtype=jnp.int32)[None, :]
        - jnp.arange(_R, dtype=jnp.int32)[:, None]
    )

    contact_out = pl.pallas_call(
        functools.partial(_contact_body, nb),
        grid=(bs, nb),
        in_specs=[
            pl.BlockSpec((1, _R, 3), lambda b, j: (b, j, 0)),
            pl.BlockSpec((1, 3, np_), lambda b, j: (b, 0, 0)),
            pl.BlockSpec((_R, np_), lambda b, j: (0, 0)),
            pl.BlockSpec((1, 1, np_), lambda b, j: (b, 0, 0)),
        ],
        out_specs=pl.BlockSpec((1, 1, 128), lambda b, j: (b, 0, 0)),
        out_shape=jax.ShapeDtypeStruct((bs, 1, 128), jnp.float32),
        scratch_shapes=[pltpu.VMEM((8, np_), jnp.float32)],
        compiler_params=pltpu.CompilerParams(
            dimension_semantics=("arbitrary", "arbitrary")
        ),
    )(pts, cols, jmat, enc)
    contactloss = contact_out[:, 0, 0]

    vcols = [vertices[bs - 1, :, d] for d in range(3)]
    fcols = [faces[:, k] for k in range(3)]
    fic_flat = faces_in_contact[bs - 1].reshape(-1)

    mesh = plsc.VectorSubcoreMesh(
        core_axis_name="c", subcore_axis_name="s", num_cores=2, num_subcores=16
    )
    fal_parts = pl.kernel(
        _fal_sc_body,
        out_type=jax.ShapeDtypeStruct((32, 16), jnp.float32),
        mesh=mesh,
        scratch_types=[
            pltpu.VMEM((32,), jnp.int32),
            pltpu.VMEM((3, 32), jnp.int32),
            pltpu.VMEM((9, 32), jnp.float32),
            pltpu.VMEM((16,), jnp.float32),
            pltpu.SemaphoreType.DMA((9,)),
        ],
    )(*vcols, *fcols, fic_flat)
    fal = jnp.sum(fal_parts[:, 0])
    face_angle_loss = jnp.zeros((bs,), dtype=vertices.dtype).at[bs - 1].set(fal)
    return (contactloss, face_angle_loss)
